# baseline (speedup 1.0000x reference)
import functools

import jax
import jax.numpy as jnp
from jax import lax
from jax.experimental import pallas as pl
from jax.experimental.pallas import tpu as pltpu
from jax.experimental.pallas import tpu_sc as plsc

_NUM_SEGMENTS = 10000
_N = 320000
_D = 128
_NW = 32
_SPW = 320
_CHUNK = 192
_NBUF = 3
_NEG = float("-inf")
_SEG_LO = tuple(312 * min(w, 30) + 320 * max(w - 30, 0) for w in range(_NW + 1))


def _make_seg_max():
    mesh = plsc.VectorSubcoreMesh(core_axis_name="c", subcore_axis_name="s")

    @functools.partial(
        pl.kernel,
        mesh=mesh,
        out_type=jax.ShapeDtypeStruct((_NUM_SEGMENTS, _D), jnp.float32),
        scratch_types=[
            pltpu.VMEM((48,), jnp.int32),
            pltpu.VMEM((_SPW + 1, _D), jnp.float32),
            pltpu.VMEM((_CHUNK, _D), jnp.float32),
            pltpu.VMEM((_CHUNK, _D), jnp.float32),
            pltpu.VMEM((_CHUNK, _D), jnp.float32),
            pltpu.VMEM((_CHUNK,), jnp.int32),
            pltpu.VMEM((_CHUNK,), jnp.int32),
            pltpu.VMEM((_CHUNK,), jnp.int32),
            pltpu.VMEM((_D,), jnp.float32),
            pltpu.SemaphoreType.DMA,
            pltpu.SemaphoreType.DMA,
            pltpu.SemaphoreType.DMA,
            pltpu.SemaphoreType.DMA,
            pltpu.SemaphoreType.DMA,
            pltpu.SemaphoreType.DMA,
        ],
    )
    def seg_max(feats_hbm, batch_hbm, bounds_hbm, out_hbm, bounds_v, acc,
                rows0, rows1, rows2, ids0, ids1, ids2, runbuf,
                sr0, sr1, sr2, si0, si1, si2):
        rowbufs = (rows0, rows1, rows2)
        idbufs = (ids0, ids1, ids2)
        srs = (sr0, sr1, sr2)
        sis = (si0, si1, si2)
        wid = lax.axis_index("s") * 2 + lax.axis_index("c")
        seg_base = 312 * wid + 8 * jnp.maximum(wid - 30, 0)
        seg_cnt = jnp.where(wid >= 30, 320, 312)

        pltpu.sync_copy(bounds_hbm, bounds_v)
        bv = bounds_v[pl.ds(wid, 16)]
        row_lo = bv[0]
        row_hi = bv[1]
        row_lo_a = (row_lo // 8) * 8
        nchunks = (row_hi - row_lo_a + _CHUNK - 1) // _CHUNK

        neg = jnp.full((16,), _NEG, jnp.float32)

        def init_body(r, _):
            for k in range(8):
                acc[r, pl.ds(k * 16, 16)] = neg
            return 0

        lax.fori_loop(0, _SPW + 1, init_body, 0)

        def flush(cur, vecs):
            for k in range(8):
                a = acc[cur, pl.ds(k * 16, 16)]
                acc[cur, pl.ds(k * 16, 16)] = jnp.maximum(a, vecs[k])

        def process(rows, ids, cur):
            def grp_body(g, cur):
                idv = ids[pl.ds(g * 16, 16)] - seg_base
                bad = (idv < 0) | (idv >= seg_cnt)
                slv = jnp.where(bad, _SPW, idv)
                uniform = (slv[0] == cur) & (slv[15] == cur)

                def fast(cur):
                    for k in range(8):
                        vs = [rows[g * 16 + r, pl.ds(k * 16, 16)]
                              for r in range(16)]
                        while len(vs) > 1:
                            vs = [jnp.maximum(vs[i], vs[i + 1])
                                  for i in range(0, len(vs), 2)]
                        a = runbuf[pl.ds(k * 16, 16)]
                        runbuf[pl.ds(k * 16, 16)] = jnp.maximum(a, vs[0])
                    return cur

                def slow(cur):
                    carry = (cur,) + tuple(
                        runbuf[pl.ds(k * 16, 16)] for k in range(8))
                    for j in range(16):
                        sid = slv[j]
                        rv = tuple(rows[g * 16 + j, pl.ds(k * 16, 16)]
                                   for k in range(8))
                        cur_j = carry[0]
                        same = sid == cur_j

                        @pl.when(jnp.logical_not(same))
                        def _(cur_j=cur_j, vecs=carry[1:]):
                            flush(cur_j, vecs)

                        vecs = tuple(
                            jnp.where(same, jnp.maximum(carry[k + 1], rv[k]),
                                      rv[k])
                            for k in range(8))
                        carry = (sid,) + vecs
                    for k in range(8):
                        runbuf[pl.ds(k * 16, 16)] = carry[k + 1]
                    return carry[0]

                return lax.cond(uniform, fast, slow, cur)

            return lax.fori_loop(0, _CHUNK // 16, grp_body, cur)

        def issue(c, rows, ids, sr, si):
            start = jnp.minimum(row_lo_a + c * _CHUNK, _N - _CHUNK)
            pltpu.async_copy(feats_hbm.at[pl.ds(start, _CHUNK)], rows, sr)
            pltpu.async_copy(batch_hbm.at[pl.ds(start, _CHUNK)], ids, si)

        def drain(rows, ids, sr, si):
            pltpu.make_async_copy(
                feats_hbm.at[pl.ds(0, _CHUNK)], rows, sr).wait()
            pltpu.make_async_copy(
                batch_hbm.at[pl.ds(0, _CHUNK)], ids, si).wait()

        nchr = jnp.maximum(((nchunks + _NBUF - 1) // _NBUF) * _NBUF, _NBUF)
        for b in range(_NBUF):
            issue(b, rowbufs[b], idbufs[b], srs[b], sis[b])

        for k in range(8):
            runbuf[pl.ds(k * 16, 16)] = neg

        def super_body(s, cur):
            for b in range(_NBUF):
                c = _NBUF * s + b
                drain(rowbufs[b], idbufs[b], srs[b], sis[b])
                cur = process(rowbufs[b], idbufs[b], cur)

                @pl.when(c + _NBUF < nchr)
                def _(b=b, c=c):
                    issue(c + _NBUF, rowbufs[b], idbufs[b], srs[b], sis[b])

            return cur

        cur = lax.fori_loop(0, nchr // _NBUF, super_body, jnp.int32(_SPW))
        flush(cur, tuple(runbuf[pl.ds(k * 16, 16)] for k in range(8)))

        def fix_body(r, _):
            for k in range(8):
                a = acc[r, pl.ds(k * 16, 16)]
                acc[r, pl.ds(k * 16, 16)] = jnp.where(a == _NEG, 0.0, a)
            return 0

        lax.fori_loop(0, _SPW, fix_body, 0)
        pltpu.sync_copy(acc.at[pl.ds(0, 312)],
                        out_hbm.at[pl.ds(seg_base, 312)])

        @pl.when(wid >= 30)
        def _():
            pltpu.sync_copy(acc.at[pl.ds(312, 8)],
                            out_hbm.at[pl.ds(seg_base + 312, 8)])

    return seg_max


_seg_max = _make_seg_max()


@jax.jit
def kernel(feats, batch):
    targets = jnp.array(_SEG_LO, dtype=jnp.int32)
    bounds = jnp.searchsorted(batch, targets, side="left").astype(jnp.int32)
    bounds = jnp.pad(bounds, (0, 48 - (_NW + 1)))
    return _seg_max(feats, batch, bounds)


# speedup vs baseline: 1.0403x; 1.0403x over previous
import functools

import jax
import jax.numpy as jnp
from jax import lax
from jax.experimental import pallas as pl
from jax.experimental.pallas import tpu as pltpu
from jax.experimental.pallas import tpu_sc as plsc

_NUM_SEGMENTS = 10000
_N = 320000
_D = 128
_NW = 32
_SPW = 320
_CHUNK = 256
_NBUF = 2
_NEG = float("-inf")
_SEG_LO = tuple(312 * min(w, 30) + 320 * max(w - 30, 0) for w in range(_NW + 1))


def _make_seg_max():
    mesh = plsc.VectorSubcoreMesh(core_axis_name="c", subcore_axis_name="s")

    @functools.partial(
        pl.kernel,
        mesh=mesh,
        out_type=jax.ShapeDtypeStruct((_NUM_SEGMENTS, _D), jnp.float32),
        scratch_types=[
            pltpu.VMEM((48,), jnp.int32),
            pltpu.VMEM((_SPW + 1, _D), jnp.float32),
            *([pltpu.VMEM((_CHUNK, _D), jnp.float32)] * _NBUF),
            *([pltpu.VMEM((_CHUNK,), jnp.int32)] * _NBUF),
            pltpu.VMEM((_D,), jnp.float32),
            *([pltpu.SemaphoreType.DMA] * (2 * _NBUF)),
        ],
    )
    def seg_max(feats_hbm, batch_hbm, bounds_hbm, out_hbm, bounds_v, acc,
                *scr):
        rowbufs = scr[:_NBUF]
        idbufs = scr[_NBUF:2 * _NBUF]
        runbuf = scr[2 * _NBUF]
        srs = scr[2 * _NBUF + 1:3 * _NBUF + 1]
        sis = scr[3 * _NBUF + 1:]
        wid = lax.axis_index("s") * 2 + lax.axis_index("c")
        seg_base = 312 * wid + 8 * jnp.maximum(wid - 30, 0)
        seg_cnt = jnp.where(wid >= 30, 320, 312)

        pltpu.sync_copy(bounds_hbm, bounds_v)
        bv = bounds_v[pl.ds(wid, 16)]
        row_lo = bv[0]
        row_hi = bv[1]
        row_lo_a = (row_lo // 8) * 8
        nchunks = (row_hi - row_lo_a + _CHUNK - 1) // _CHUNK

        neg = jnp.full((16,), _NEG, jnp.float32)

        def init_body(r, _):
            for k in range(8):
                acc[r, pl.ds(k * 16, 16)] = neg
            return 0

        lax.fori_loop(0, _SPW + 1, init_body, 0)

        def flush(cur, vecs):
            for k in range(8):
                a = acc[cur, pl.ds(k * 16, 16)]
                acc[cur, pl.ds(k * 16, 16)] = jnp.maximum(a, vecs[k])

        def process(rows, ids, cur):
            def grp_body(g, cur):
                idv = ids[pl.ds(g * 16, 16)] - seg_base
                bad = (idv < 0) | (idv >= seg_cnt)
                slv = jnp.where(bad, _SPW, idv)
                uniform = (slv[0] == cur) & (slv[15] == cur)

                def fast(cur):
                    for k in range(8):
                        vs = [rows[g * 16 + r, pl.ds(k * 16, 16)]
                              for r in range(16)]
                        while len(vs) > 1:
                            vs = [jnp.maximum(vs[i], vs[i + 1])
                                  for i in range(0, len(vs), 2)]
                        a = runbuf[pl.ds(k * 16, 16)]
                        runbuf[pl.ds(k * 16, 16)] = jnp.maximum(a, vs[0])
                    return cur

                def slow(cur):
                    carry = (cur,) + tuple(
                        runbuf[pl.ds(k * 16, 16)] for k in range(8))
                    for j in range(16):
                        sid = slv[j]
                        rv = tuple(rows[g * 16 + j, pl.ds(k * 16, 16)]
                                   for k in range(8))
                        cur_j = carry[0]
                        same = sid == cur_j

                        @pl.when(jnp.logical_not(same))
                        def _(cur_j=cur_j, vecs=carry[1:]):
                            flush(cur_j, vecs)

                        vecs = tuple(
                            jnp.where(same, jnp.maximum(carry[k + 1], rv[k]),
                                      rv[k])
                            for k in range(8))
                        carry = (sid,) + vecs
                    for k in range(8):
                        runbuf[pl.ds(k * 16, 16)] = carry[k + 1]
                    return carry[0]

                return lax.cond(uniform, fast, slow, cur)

            return lax.fori_loop(0, _CHUNK // 16, grp_body, cur)

        def issue(c, rows, ids, sr, si):
            start = jnp.minimum(row_lo_a + c * _CHUNK, _N - _CHUNK)
            pltpu.async_copy(feats_hbm.at[pl.ds(start, _CHUNK)], rows, sr)
            pltpu.async_copy(batch_hbm.at[pl.ds(start, _CHUNK)], ids, si)

        def drain(rows, ids, sr, si):
            pltpu.make_async_copy(
                feats_hbm.at[pl.ds(0, _CHUNK)], rows, sr).wait()
            pltpu.make_async_copy(
                batch_hbm.at[pl.ds(0, _CHUNK)], ids, si).wait()

        nchr = jnp.maximum(((nchunks + _NBUF - 1) // _NBUF) * _NBUF, _NBUF)
        for b in range(_NBUF):
            issue(b, rowbufs[b], idbufs[b], srs[b], sis[b])

        for k in range(8):
            runbuf[pl.ds(k * 16, 16)] = neg

        def super_body(s, cur):
            for b in range(_NBUF):
                c = _NBUF * s + b
                drain(rowbufs[b], idbufs[b], srs[b], sis[b])
                cur = process(rowbufs[b], idbufs[b], cur)

                @pl.when(c + _NBUF < nchr)
                def _(b=b, c=c):
                    issue(c + _NBUF, rowbufs[b], idbufs[b], srs[b], sis[b])

            return cur

        cur = lax.fori_loop(0, nchr // _NBUF, super_body, jnp.int32(_SPW))
        flush(cur, tuple(runbuf[pl.ds(k * 16, 16)] for k in range(8)))

        def fix_body(r, _):
            for k in range(8):
                a = acc[r, pl.ds(k * 16, 16)]
                acc[r, pl.ds(k * 16, 16)] = jnp.where(a == _NEG, 0.0, a)
            return 0

        lax.fori_loop(0, _SPW, fix_body, 0)
        pltpu.sync_copy(acc.at[pl.ds(0, 312)],
                        out_hbm.at[pl.ds(seg_base, 312)])

        @pl.when(wid >= 30)
        def _():
            pltpu.sync_copy(acc.at[pl.ds(312, 8)],
                            out_hbm.at[pl.ds(seg_base + 312, 8)])

    return seg_max


_seg_max = _make_seg_max()


@jax.jit
def kernel(feats, batch):
    targets = jnp.array(_SEG_LO, dtype=jnp.int32)
    bounds = jnp.searchsorted(batch, targets, side="left").astype(jnp.int32)
    bounds = jnp.pad(bounds, (0, 48 - (_NW + 1)))
    return _seg_max(feats, batch, bounds)
